# baseline (speedup 1.0000x reference)
import functools

import jax
import jax.numpy as jnp
from jax import lax
from jax.experimental import pallas as pl
from jax.experimental.pallas import tpu as pltpu
from jax.experimental.pallas import tpu_sc as plsc

_NUM_USERS = 100000
_MAX_MEM = 16
_D = 64
_TOP_K = 4
_INV_TEMP = 10.0
_BATCH = 4096
_BLK = _MAX_MEM * _D

_NC = 2
_NS = 16
_NW = _NC * _NS
_BPW = _BATCH // _NW
_CH = 32
_NCHUNK = _BPW // _CH

_NFULL = _NUM_USERS // 128
_TAIL0 = _NFULL * 128
_TAIL = _NUM_USERS - _TAIL0
_NUNITS = _NFULL + 1
_UPT = -(-_NUNITS // _NW)

_SC_PARAMS = pltpu.CompilerParams(use_tc_tiling_on_sc=True,
                                  needs_layout_passes=False,
                                  disable_bounds_checks=True)


def _extract_body(keysT, valsT, uid_hbm, uk_hbm, uv_hbm,
                  uid_v, qlist_v, ulist_v, chunk_v, tail_v, stage_v, sem):
    w = lax.axis_index("s") * _NC + lax.axis_index("c")
    pltpu.sync_copy(uid_hbm, uid_v)
    iota = lax.iota(jnp.int32, 16)

    def do_window(c, base, width, cbuf, is_tail):
        def scan_body(g, mcnt):
            u16 = uid_v[pl.ds(g * 16, 16)]
            if is_tail:
                m = u16 >= _TAIL0
            else:
                m = lax.shift_right_logical(u16, 7) == c
            plsc.store_compressed(qlist_v.at[pl.ds(mcnt, 16)], iota + g * 16,
                                  mask=m)
            plsc.store_compressed(ulist_v.at[pl.ds(mcnt, 16)], u16, mask=m)
            return mcnt + plsc.all_reduce_population_count(m)[0]

        mcnt = lax.fori_loop(0, _BATCH // 16, scan_body, 0)

        for src, dst in ((keysT, uk_hbm), (valsT, uv_hbm)):
            for qt in range(4):
                pltpu.async_copy(
                    src.at[pl.ds(qt * 4, 4), :, pl.ds(base, width)],
                    cbuf, sem).wait()

                def q_body(qi, carry):
                    qsp = jnp.broadcast_to(qi, (16,)).astype(jnp.int32)
                    qg = plsc.load_gather(qlist_v, [qsp])[0]
                    uu = plsc.load_gather(ulist_v, [qsp])[0]
                    colv = jnp.broadcast_to(uu - base, (16,))
                    for r in range(_BLK // 4 // 16):
                        fl = iota + r * 16
                        i0 = lax.shift_right_logical(fl, 6)
                        i1 = jnp.bitwise_and(fl, 63)
                        stage_v[pl.ds(r * 16, 16)] = plsc.load_gather(
                            cbuf, [i0, i1, colv])
                    pltpu.sync_copy(
                        stage_v,
                        dst.at[pl.ds(qg * _BLK + qt * (_BLK // 4),
                                     _BLK // 4)])
                    return carry

                lax.fori_loop(0, mcnt, q_body, 0)

    def unit_body(i, carry):
        c = w + i * _NW

        @pl.when(c < _NFULL)
        def _():
            do_window(c, c * 128, 128, chunk_v, False)

        @pl.when(c == _NFULL)
        def _():
            do_window(c, _TAIL0, _TAIL, tail_v, True)

        return carry

    lax.fori_loop(0, _UPT, unit_body, 0)


_extract = functools.partial(
    pl.kernel,
    out_type=(jax.ShapeDtypeStruct((_BATCH * _BLK,), jnp.float32),
              jax.ShapeDtypeStruct((_BATCH * _BLK,), jnp.float32)),
    mesh=plsc.VectorSubcoreMesh(core_axis_name="c", subcore_axis_name="s"),
    compiler_params=_SC_PARAMS,
    scratch_types=[
        pltpu.VMEM((_BATCH,), jnp.int32),
        pltpu.VMEM((_BATCH + 16,), jnp.int32),
        pltpu.VMEM((_BATCH + 16,), jnp.int32),
        pltpu.VMEM((4, _D, 128), jnp.float32),
        pltpu.VMEM((4, _D, _TAIL), jnp.float32),
        pltpu.VMEM((_BLK // 4,), jnp.float32),
        pltpu.SemaphoreType.DMA,
    ],
)(_extract_body)


def _fast_rsqrt(x):
    i = plsc.bitcast(x, jnp.int32)
    i = jnp.int32(0x5F3759DF) - lax.shift_right_logical(i, 1)
    r = plsc.bitcast(i, jnp.float32)
    for _ in range(3):
        r = r * (1.5 - 0.5 * x * r * r)
    return r


def _retrieve_body(qn_hbm, uk_hbm, uv_hbm, uid_hbm, cnt_hbm, out_hbm,
                   uid_v, cnt_v, qn_v, keys_v, vals_v, out_v, sem, sem2):
    wid = lax.axis_index("s") * _NC + lax.axis_index("c")
    base = wid * _BPW

    pltpu.sync_copy(uid_hbm.at[pl.ds(base, _BPW)], uid_v)
    pltpu.sync_copy(qn_hbm.at[pl.ds(base * _D, _BPW * _D)], qn_v)
    pltpu.async_copy(cnt_hbm.at[uid_v], cnt_v, sem).wait()

    iota = lax.iota(jnp.int32, 16)
    first4 = iota < _TOP_K

    for ci in range(_NCHUNK):
        off0 = (base + ci * _CH) * _BLK
        kcp = pltpu.async_copy(uk_hbm.at[pl.ds(off0, _CH * _BLK)], keys_v, sem)
        vcp = pltpu.async_copy(uv_hbm.at[pl.ds(off0, _CH * _BLK)], vals_v,
                               sem2)
        kcp.wait()
        vcp.wait()

        def q_body(q, carry, ci=ci):
            qq = ci * _CH + q
            qsplat = jnp.broadcast_to(qq, (16,)).astype(jnp.int32)
            dot = jnp.zeros((16,), jnp.float32)
            nrm = jnp.zeros((16,), jnp.float32)
            qoff = qq * _D
            kbase = iota * _D + q * _BLK
            for r in range(_D // 16):
                qblk = qn_v[pl.ds(qoff + 16 * r, 16)]
                for j in range(16):
                    kcol = plsc.load_gather(keys_v, [kbase + (16 * r + j)])
                    dot = dot + kcol * qblk[j]
                    nrm = nrm + kcol * kcol
            cntv = plsc.load_gather(cnt_v, [qsplat])
            sims = dot * _fast_rsqrt(jnp.maximum(nrm, 1e-24))
            msims = jnp.where(iota < cntv, sims, jnp.float32(-1e9))
            vmax = jnp.max(msims)
            sk, sv = plsc.sort_key_val(msims, iota, descending=True)
            e = jnp.where(first4, jnp.exp((sk - vmax) * _INV_TEMP), 0.0)
            w = e / jnp.sum(e)
            accs = [jnp.zeros((16,), jnp.float32) for _ in range(_D // 16)]
            for k in range(_TOP_K):
                slot = jnp.broadcast_to(sv[k], (16,)).astype(jnp.int32)
                wk = w[k]
                for r in range(_D // 16):
                    vrow = plsc.load_gather(
                        vals_v, [slot * _D + (q * _BLK + 16 * r) + iota])
                    accs[r] = accs[r] + wk * vrow
            for r in range(_D // 16):
                out_v[pl.ds(q * _D + 16 * r, 16)] = accs[r]
            return carry

        lax.fori_loop(0, _CH, q_body, 0)

        pltpu.sync_copy(out_v,
                        out_hbm.at[pl.ds((base + ci * _CH) * _D, _CH * _D)])


_retrieve = functools.partial(
    pl.kernel,
    out_type=jax.ShapeDtypeStruct((_BATCH * _D,), jnp.float32),
    mesh=plsc.VectorSubcoreMesh(core_axis_name="c", subcore_axis_name="s"),
    compiler_params=_SC_PARAMS,
    scratch_types=[
        pltpu.VMEM((_BPW,), jnp.int32),
        pltpu.VMEM((_BPW,), jnp.int32),
        pltpu.VMEM((_BPW * _D,), jnp.float32),
        pltpu.VMEM((_CH * _BLK,), jnp.float32),
        pltpu.VMEM((_CH * _BLK,), jnp.float32),
        pltpu.VMEM((_CH * _D,), jnp.float32),
        pltpu.SemaphoreType.DMA,
        pltpu.SemaphoreType.DMA,
    ],
)(_retrieve_body)


def _qn_body(q_ref, wk_ref, o_ref):
    y = lax.dot_general(q_ref[...], wk_ref[...], (((1,), (1,)), ((), ())),
                        preferred_element_type=jnp.float32)
    n = jnp.sqrt(jnp.sum(y * y, axis=-1, keepdims=True))
    o_ref[...] = y / jnp.maximum(n, 1e-12)


_qn_call = pl.pallas_call(
    _qn_body,
    out_shape=jax.ShapeDtypeStruct((_BATCH, _D), jnp.float32),
)


def _proj_body(b_ref, wv_ref, o_ref):
    o_ref[...] = lax.dot_general(b_ref[...], wv_ref[...],
                                 (((1,), (1,)), ((), ())),
                                 preferred_element_type=jnp.float32)


_proj_call = pl.pallas_call(
    _proj_body,
    out_shape=jax.ShapeDtypeStruct((_BATCH, _D), jnp.float32),
)


def kernel(query, keys_buf, values_buf, W_key, W_val, episodic_scale,
           user_ids, memory_count):
    qn = _qn_call(query, W_key)
    uid = user_ids.astype(jnp.int32)
    cnt = memory_count.astype(jnp.int32)
    keysT = jnp.transpose(keys_buf, (1, 2, 0))
    valsT = jnp.transpose(values_buf, (1, 2, 0))
    uk, uv = _extract(keysT, valsT, uid)
    blended = _retrieve(qn.reshape(-1), uk, uv, uid, cnt)
    blended = blended.reshape(_BATCH, _D)
    return _proj_call(blended, W_val * episodic_scale)


# speedup vs baseline: 1.4966x; 1.4966x over previous
import functools

import jax
import jax.numpy as jnp
from jax import lax
from jax.experimental import pallas as pl
from jax.experimental.pallas import tpu as pltpu
from jax.experimental.pallas import tpu_sc as plsc

_NUM_USERS = 100000
_MAX_MEM = 16
_D = 64
_TOP_K = 4
_INV_TEMP = 10.0
_BATCH = 4096
_BLK = _MAX_MEM * _D

_NC = 2
_NS = 16
_NW = _NC * _NS
_BPW = _BATCH // _NW
_CH = 32
_NCHUNK = _BPW // _CH

_NFULL = _NUM_USERS // 128
_TAIL0 = _NFULL * 128
_TAIL = _NUM_USERS - _TAIL0
_NUNITS = _NFULL + 1
_UPT = -(-_NUNITS // _NW)

_SC_PARAMS = pltpu.CompilerParams(use_tc_tiling_on_sc=True,
                                  needs_layout_passes=False,
                                  disable_bounds_checks=True)


def _extract_body(keysT, valsT, uid_hbm, uk_hbm, uv_hbm,
                  uid_v, plist_v, chunk_a, chunk_b, tail_v,
                  stage_v, sem_a, sem_b):
    w = lax.axis_index("s") * _NC + lax.axis_index("c")
    pltpu.sync_copy(uid_hbm, uid_v)
    iota = lax.iota(jnp.int32, 16)

    def scan(c, is_tail):
        def scan_body(g, mcnt):
            u16 = uid_v[pl.ds(g * 16, 16)]
            if is_tail:
                m = u16 >= _TAIL0
            else:
                m = lax.shift_right_logical(u16, 7) == c
            plsc.store_compressed(plist_v.at[pl.ds(mcnt, 16)],
                                  u16 * 4096 + (iota + g * 16), mask=m)
            return mcnt + plsc.all_reduce_population_count(m)[0]

        return lax.fori_loop(0, _BATCH // 16, scan_body, 0)

    def do_window(c, base):
        bufs = (chunk_a, chunk_b)
        sems = (sem_a, sem_b)
        items = [(keysT, uk_hbm, qt) for qt in range(4)] + \
                [(valsT, uv_hbm, qt) for qt in range(4)]

        def issue(j):
            src, _, qt = items[j]
            return pltpu.async_copy(
                src.at[pl.ds(qt * 4, 4), :, pl.ds(base, 128)],
                bufs[j % 2], sems[j % 2])

        handles = [issue(0)]
        mcnt = scan(c, False)

        for j in range(8):
            handles[j].wait()
            if j + 1 < 8:
                handles.append(issue(j + 1))
            _, dst, qt = items[j]
            cbuf = bufs[j % 2]

            def q_body(qi, carry, cbuf=cbuf, dst=dst, qt=qt):
                qsp = jnp.broadcast_to(qi, (16,)).astype(jnp.int32)
                pk = plsc.load_gather(plist_v, [qsp])[0]
                qg = jnp.bitwise_and(pk, 4095)
                uu = lax.shift_right_logical(pk, 12)
                colv = jnp.broadcast_to(uu - base, (16,))
                for r in range(_BLK // 4 // 16):
                    fl = iota + r * 16
                    i0 = lax.shift_right_logical(fl, 6)
                    i1 = jnp.bitwise_and(fl, 63)
                    stage_v[pl.ds(r * 16, 16)] = plsc.load_gather(
                        cbuf, [i0, i1, colv])
                pltpu.sync_copy(
                    stage_v,
                    dst.at[pl.ds(qg * _BLK + qt * (_BLK // 4),
                                 _BLK // 4)])
                return carry

            lax.fori_loop(0, mcnt, q_body, 0)

    def do_tail():
        mcnt = scan(0, True)
        for src, dst in ((keysT, uk_hbm), (valsT, uv_hbm)):
            for et in range(8):
                pltpu.async_copy(
                    src.at[pl.ds(et * 2, 2), :, pl.ds(_TAIL0, _TAIL)],
                    tail_v, sem_a).wait()

                def q_body(qi, carry, dst=dst, et=et):
                    qsp = jnp.broadcast_to(qi, (16,)).astype(jnp.int32)
                    pk = plsc.load_gather(plist_v, [qsp])[0]
                    qg = jnp.bitwise_and(pk, 4095)
                    uu = lax.shift_right_logical(pk, 12)
                    colv = jnp.broadcast_to(uu - _TAIL0, (16,))
                    for r in range(_BLK // 8 // 16):
                        fl = iota + r * 16
                        i0 = lax.shift_right_logical(fl, 6)
                        i1 = jnp.bitwise_and(fl, 63)
                        stage_v[pl.ds(r * 16, 16)] = plsc.load_gather(
                            tail_v, [i0, i1, colv])
                    pltpu.sync_copy(
                        stage_v.at[pl.ds(0, _BLK // 8)],
                        dst.at[pl.ds(qg * _BLK + et * (_BLK // 8),
                                     _BLK // 8)])
                    return carry

                lax.fori_loop(0, mcnt, q_body, 0)

    def unit_body(i, carry):
        c = w + i * _NW

        @pl.when(c < _NFULL)
        def _():
            do_window(c, c * 128)

        @pl.when(c == _NFULL)
        def _():
            do_tail()

        return carry

    lax.fori_loop(0, _UPT, unit_body, 0)


_extract = functools.partial(
    pl.kernel,
    out_type=(jax.ShapeDtypeStruct((_BATCH * _BLK,), jnp.float32),
              jax.ShapeDtypeStruct((_BATCH * _BLK,), jnp.float32)),
    mesh=plsc.VectorSubcoreMesh(core_axis_name="c", subcore_axis_name="s"),
    compiler_params=_SC_PARAMS,
    scratch_types=[
        pltpu.VMEM((_BATCH,), jnp.int32),
        pltpu.VMEM((_BATCH + 16,), jnp.int32),
        pltpu.VMEM((4, _D, 128), jnp.float32),
        pltpu.VMEM((4, _D, 128), jnp.float32),
        pltpu.VMEM((2, _D, _TAIL), jnp.float32),
        pltpu.VMEM((_BLK // 4,), jnp.float32),
        pltpu.SemaphoreType.DMA,
        pltpu.SemaphoreType.DMA,
    ],
)(_extract_body)


def _fast_rsqrt(x):
    i = plsc.bitcast(x, jnp.int32)
    i = jnp.int32(0x5F3759DF) - lax.shift_right_logical(i, 1)
    r = plsc.bitcast(i, jnp.float32)
    for _ in range(3):
        r = r * (1.5 - 0.5 * x * r * r)
    return r


def _retrieve_body(qn_hbm, uk_hbm, uv_hbm, uid_hbm, cnt_hbm, out_hbm,
                   uid_v, cnt_v, qn_v, keys_v, vals_v, out_v, sem, sem2):
    wid = lax.axis_index("s") * _NC + lax.axis_index("c")
    base = wid * _BPW

    pltpu.sync_copy(uid_hbm.at[pl.ds(base, _BPW)], uid_v)
    pltpu.sync_copy(qn_hbm.at[pl.ds(base * _D, _BPW * _D)], qn_v)
    pltpu.async_copy(cnt_hbm.at[uid_v], cnt_v, sem).wait()

    iota = lax.iota(jnp.int32, 16)
    first4 = iota < _TOP_K

    for ci in range(_NCHUNK):
        off0 = (base + ci * _CH) * _BLK
        kcp = pltpu.async_copy(uk_hbm.at[pl.ds(off0, _CH * _BLK)], keys_v, sem)
        vcp = pltpu.async_copy(uv_hbm.at[pl.ds(off0, _CH * _BLK)], vals_v,
                               sem2)
        kcp.wait()
        vcp.wait()

        def q_body(q, carry, ci=ci):
            qq = ci * _CH + q
            qsplat = jnp.broadcast_to(qq, (16,)).astype(jnp.int32)
            dot = jnp.zeros((16,), jnp.float32)
            nrm = jnp.zeros((16,), jnp.float32)
            qoff = qq * _D
            kbase = iota * _D + q * _BLK
            for r in range(_D // 16):
                qblk = qn_v[pl.ds(qoff + 16 * r, 16)]
                for j in range(16):
                    kcol = plsc.load_gather(keys_v, [kbase + (16 * r + j)])
                    dot = dot + kcol * qblk[j]
                    nrm = nrm + kcol * kcol
            cntv = plsc.load_gather(cnt_v, [qsplat])
            sims = dot * _fast_rsqrt(jnp.maximum(nrm, 1e-24))
            msims = jnp.where(iota < cntv, sims, jnp.float32(-1e9))
            vmax = jnp.max(msims)
            sk, sv = plsc.sort_key_val(msims, iota, descending=True)
            e = jnp.where(first4, jnp.exp((sk - vmax) * _INV_TEMP), 0.0)
            w = e / jnp.sum(e)
            accs = [jnp.zeros((16,), jnp.float32) for _ in range(_D // 16)]
            for k in range(_TOP_K):
                slot = jnp.broadcast_to(sv[k], (16,)).astype(jnp.int32)
                wk = w[k]
                for r in range(_D // 16):
                    vrow = plsc.load_gather(
                        vals_v, [slot * _D + (q * _BLK + 16 * r) + iota])
                    accs[r] = accs[r] + wk * vrow
            for r in range(_D // 16):
                out_v[pl.ds(q * _D + 16 * r, 16)] = accs[r]
            return carry

        lax.fori_loop(0, _CH, q_body, 0)

        pltpu.sync_copy(out_v,
                        out_hbm.at[pl.ds((base + ci * _CH) * _D, _CH * _D)])


_retrieve = functools.partial(
    pl.kernel,
    out_type=jax.ShapeDtypeStruct((_BATCH * _D,), jnp.float32),
    mesh=plsc.VectorSubcoreMesh(core_axis_name="c", subcore_axis_name="s"),
    compiler_params=_SC_PARAMS,
    scratch_types=[
        pltpu.VMEM((_BPW,), jnp.int32),
        pltpu.VMEM((_BPW,), jnp.int32),
        pltpu.VMEM((_BPW * _D,), jnp.float32),
        pltpu.VMEM((_CH * _BLK,), jnp.float32),
        pltpu.VMEM((_CH * _BLK,), jnp.float32),
        pltpu.VMEM((_CH * _D,), jnp.float32),
        pltpu.SemaphoreType.DMA,
        pltpu.SemaphoreType.DMA,
    ],
)(_retrieve_body)


def _qn_body(q_ref, wk_ref, o_ref):
    y = lax.dot_general(q_ref[...], wk_ref[...], (((1,), (1,)), ((), ())),
                        preferred_element_type=jnp.float32)
    n = jnp.sqrt(jnp.sum(y * y, axis=-1, keepdims=True))
    o_ref[...] = y / jnp.maximum(n, 1e-12)


_qn_call = pl.pallas_call(
    _qn_body,
    out_shape=jax.ShapeDtypeStruct((_BATCH, _D), jnp.float32),
)


def _proj_body(b_ref, wv_ref, o_ref):
    o_ref[...] = lax.dot_general(b_ref[...], wv_ref[...],
                                 (((1,), (1,)), ((), ())),
                                 preferred_element_type=jnp.float32)


_proj_call = pl.pallas_call(
    _proj_body,
    out_shape=jax.ShapeDtypeStruct((_BATCH, _D), jnp.float32),
)


def kernel(query, keys_buf, values_buf, W_key, W_val, episodic_scale,
           user_ids, memory_count):
    qn = _qn_call(query, W_key)
    uid = user_ids.astype(jnp.int32)
    cnt = memory_count.astype(jnp.int32)
    keysT = jnp.transpose(keys_buf, (1, 2, 0))
    valsT = jnp.transpose(values_buf, (1, 2, 0))
    uk, uv = _extract(keysT, valsT, uid)
    blended = _retrieve(qn.reshape(-1), uk, uv, uid, cnt)
    blended = blended.reshape(_BATCH, _D)
    return _proj_call(blended, W_val * episodic_scale)


# speedup vs baseline: 1.5306x; 1.0227x over previous
import functools

import jax
import jax.numpy as jnp
from jax import lax
from jax.experimental import pallas as pl
from jax.experimental.pallas import tpu as pltpu
from jax.experimental.pallas import tpu_sc as plsc

_NUM_USERS = 100000
_MAX_MEM = 16
_D = 64
_TOP_K = 4
_INV_TEMP = 10.0
_BATCH = 4096
_BLK = _MAX_MEM * _D

_NC = 2
_NS = 16
_NW = _NC * _NS
_BPW = _BATCH // _NW
_CH = 32
_NCHUNK = _BPW // _CH

_NFULL = _NUM_USERS // 128
_TAIL0 = _NFULL * 128
_TAIL = _NUM_USERS - _TAIL0
_NUNITS = _NFULL + 1
_UPT = -(-_NUNITS // _NW)

_SC_PARAMS = pltpu.CompilerParams(use_tc_tiling_on_sc=True,
                                  needs_layout_passes=False,
                                  disable_bounds_checks=True)


def _extract_body(keysT, valsT, uid_hbm, uk_hbm, uv_hbm,
                  uid_v, plist_v, chunk_a, chunk_b, tail_v,
                  stage_v, stage_w, sem_a, sem_b, sem_s0, sem_s1):
    w = lax.axis_index("s") * _NC + lax.axis_index("c")
    pltpu.sync_copy(uid_hbm, uid_v)
    iota = lax.iota(jnp.int32, 16)

    def scan(c, is_tail):
        def scan_body(g, mcnt):
            u16 = uid_v[pl.ds(g * 16, 16)]
            if is_tail:
                m = u16 >= _TAIL0
            else:
                m = lax.shift_right_logical(u16, 7) == c
            plsc.store_compressed(plist_v.at[pl.ds(mcnt, 16)],
                                  u16 * 4096 + (iota + g * 16), mask=m)
            return mcnt + plsc.all_reduce_population_count(m)[0]

        return lax.fori_loop(0, _BATCH // 16, scan_body, 0)

    def do_window(c, base):
        bufs = (chunk_a, chunk_b)
        sems = (sem_a, sem_b)
        items = [(keysT, uk_hbm, qt) for qt in range(4)] + \
                [(valsT, uv_hbm, qt) for qt in range(4)]

        def issue(j):
            src, _, qt = items[j]
            return pltpu.async_copy(
                src.at[pl.ds(qt * 4, 4), :, pl.ds(base, 128)],
                bufs[j % 2], sems[j % 2])

        handles = [issue(0)]
        mcnt = scan(c, False)

        for j in range(8):
            handles[j].wait()
            if j + 1 < 8:
                handles.append(issue(j + 1))
            _, dst, qt = items[j]
            cbuf = bufs[j % 2]
            stages = (stage_v, stage_w)
            ssems = (sem_s0, sem_s1)

            def q_body(qi, carry, cbuf=cbuf, dst=dst, qt=qt):
                qsp = jnp.broadcast_to(qi, (16,)).astype(jnp.int32)
                pk = plsc.load_gather(plist_v, [qsp])[0]
                qg = jnp.bitwise_and(pk, 4095)
                uu = lax.shift_right_logical(pk, 12)
                colv = jnp.broadcast_to(uu - base, (16,))
                doff = qg * _BLK + qt * (_BLK // 4)
                for par in range(2):
                    @pl.when(jnp.bitwise_and(qi, 1) == par)
                    def _(par=par):
                        st = stages[par]
                        ss = ssems[par]

                        @pl.when(qi >= 2)
                        def _():
                            pltpu.make_async_copy(
                                st, dst.at[pl.ds(0, _BLK // 4)], ss).wait()

                        for r in range(_BLK // 4 // 16):
                            fl = iota + r * 16
                            i0 = lax.shift_right_logical(fl, 6)
                            i1 = jnp.bitwise_and(fl, 63)
                            st[pl.ds(r * 16, 16)] = plsc.load_gather(
                                cbuf, [i0, i1, colv])
                        pltpu.async_copy(st, dst.at[pl.ds(doff, _BLK // 4)],
                                         ss)
                return carry

            lax.fori_loop(0, mcnt, q_body, 0)

            @pl.when(mcnt >= 1)
            def _(dst=dst):
                pltpu.make_async_copy(
                    stages[0], dst.at[pl.ds(0, _BLK // 4)],
                    ssems[0]).wait()

            @pl.when(mcnt >= 2)
            def _(dst=dst):
                pltpu.make_async_copy(
                    stages[1], dst.at[pl.ds(0, _BLK // 4)],
                    ssems[1]).wait()

    def do_tail():
        mcnt = scan(0, True)
        for src, dst in ((keysT, uk_hbm), (valsT, uv_hbm)):
            for et in range(8):
                pltpu.async_copy(
                    src.at[pl.ds(et * 2, 2), :, pl.ds(_TAIL0, _TAIL)],
                    tail_v, sem_a).wait()

                def q_body(qi, carry, dst=dst, et=et):
                    qsp = jnp.broadcast_to(qi, (16,)).astype(jnp.int32)
                    pk = plsc.load_gather(plist_v, [qsp])[0]
                    qg = jnp.bitwise_and(pk, 4095)
                    uu = lax.shift_right_logical(pk, 12)
                    colv = jnp.broadcast_to(uu - _TAIL0, (16,))
                    for r in range(_BLK // 8 // 16):
                        fl = iota + r * 16
                        i0 = lax.shift_right_logical(fl, 6)
                        i1 = jnp.bitwise_and(fl, 63)
                        stage_v[pl.ds(r * 16, 16)] = plsc.load_gather(
                            tail_v, [i0, i1, colv])
                    pltpu.sync_copy(
                        stage_v.at[pl.ds(0, _BLK // 8)],
                        dst.at[pl.ds(qg * _BLK + et * (_BLK // 8),
                                     _BLK // 8)])
                    return carry

                lax.fori_loop(0, mcnt, q_body, 0)

    def unit_body(i, carry):
        c = w + i * _NW

        @pl.when(c < _NFULL)
        def _():
            do_window(c, c * 128)

        @pl.when(c == _NFULL)
        def _():
            do_tail()

        return carry

    lax.fori_loop(0, _UPT, unit_body, 0)


_extract = functools.partial(
    pl.kernel,
    out_type=(jax.ShapeDtypeStruct((_BATCH * _BLK,), jnp.float32),
              jax.ShapeDtypeStruct((_BATCH * _BLK,), jnp.float32)),
    mesh=plsc.VectorSubcoreMesh(core_axis_name="c", subcore_axis_name="s"),
    compiler_params=_SC_PARAMS,
    scratch_types=[
        pltpu.VMEM((_BATCH,), jnp.int32),
        pltpu.VMEM((_BATCH + 16,), jnp.int32),
        pltpu.VMEM((4, _D, 128), jnp.float32),
        pltpu.VMEM((4, _D, 128), jnp.float32),
        pltpu.VMEM((2, _D, _TAIL), jnp.float32),
        pltpu.VMEM((_BLK // 4,), jnp.float32),
        pltpu.VMEM((_BLK // 4,), jnp.float32),
        pltpu.SemaphoreType.DMA,
        pltpu.SemaphoreType.DMA,
        pltpu.SemaphoreType.DMA,
        pltpu.SemaphoreType.DMA,
    ],
)(_extract_body)


def _fast_rsqrt(x):
    i = plsc.bitcast(x, jnp.int32)
    i = jnp.int32(0x5F3759DF) - lax.shift_right_logical(i, 1)
    r = plsc.bitcast(i, jnp.float32)
    for _ in range(3):
        r = r * (1.5 - 0.5 * x * r * r)
    return r


def _retrieve_body(qn_hbm, uk_hbm, uv_hbm, uid_hbm, cnt_hbm, out_hbm,
                   uid_v, cnt_v, qn_v, keys_v, vals_v, out_v, sem, sem2):
    wid = lax.axis_index("s") * _NC + lax.axis_index("c")
    base = wid * _BPW

    pltpu.sync_copy(uid_hbm.at[pl.ds(base, _BPW)], uid_v)
    pltpu.sync_copy(qn_hbm.at[pl.ds(base * _D, _BPW * _D)], qn_v)
    pltpu.async_copy(cnt_hbm.at[uid_v], cnt_v, sem).wait()

    iota = lax.iota(jnp.int32, 16)
    first4 = iota < _TOP_K

    for ci in range(_NCHUNK):
        off0 = (base + ci * _CH) * _BLK
        kcp = pltpu.async_copy(uk_hbm.at[pl.ds(off0, _CH * _BLK)], keys_v, sem)
        vcp = pltpu.async_copy(uv_hbm.at[pl.ds(off0, _CH * _BLK)], vals_v,
                               sem2)
        kcp.wait()
        vcp.wait()

        def q_body(q, carry, ci=ci):
            qq = ci * _CH + q
            qsplat = jnp.broadcast_to(qq, (16,)).astype(jnp.int32)
            dot = jnp.zeros((16,), jnp.float32)
            nrm = jnp.zeros((16,), jnp.float32)
            qoff = qq * _D
            kbase = iota * _D + q * _BLK
            for r in range(_D // 16):
                qblk = qn_v[pl.ds(qoff + 16 * r, 16)]
                for j in range(16):
                    kcol = plsc.load_gather(keys_v, [kbase + (16 * r + j)])
                    dot = dot + kcol * qblk[j]
                    nrm = nrm + kcol * kcol
            cntv = plsc.load_gather(cnt_v, [qsplat])
            sims = dot * _fast_rsqrt(jnp.maximum(nrm, 1e-24))
            msims = jnp.where(iota < cntv, sims, jnp.float32(-1e9))
            vmax = jnp.max(msims)
            sk, sv = plsc.sort_key_val(msims, iota, descending=True)
            e = jnp.where(first4, jnp.exp((sk - vmax) * _INV_TEMP), 0.0)
            w = e / jnp.sum(e)
            accs = [jnp.zeros((16,), jnp.float32) for _ in range(_D // 16)]
            for k in range(_TOP_K):
                slot = jnp.broadcast_to(sv[k], (16,)).astype(jnp.int32)
                wk = w[k]
                for r in range(_D // 16):
                    vrow = plsc.load_gather(
                        vals_v, [slot * _D + (q * _BLK + 16 * r) + iota])
                    accs[r] = accs[r] + wk * vrow
            for r in range(_D // 16):
                out_v[pl.ds(q * _D + 16 * r, 16)] = accs[r]
            return carry

        lax.fori_loop(0, _CH, q_body, 0)

        pltpu.sync_copy(out_v,
                        out_hbm.at[pl.ds((base + ci * _CH) * _D, _CH * _D)])


_retrieve = functools.partial(
    pl.kernel,
    out_type=jax.ShapeDtypeStruct((_BATCH * _D,), jnp.float32),
    mesh=plsc.VectorSubcoreMesh(core_axis_name="c", subcore_axis_name="s"),
    compiler_params=_SC_PARAMS,
    scratch_types=[
        pltpu.VMEM((_BPW,), jnp.int32),
        pltpu.VMEM((_BPW,), jnp.int32),
        pltpu.VMEM((_BPW * _D,), jnp.float32),
        pltpu.VMEM((_CH * _BLK,), jnp.float32),
        pltpu.VMEM((_CH * _BLK,), jnp.float32),
        pltpu.VMEM((_CH * _D,), jnp.float32),
        pltpu.SemaphoreType.DMA,
        pltpu.SemaphoreType.DMA,
    ],
)(_retrieve_body)


def _qn_body(q_ref, wk_ref, o_ref):
    y = lax.dot_general(q_ref[...], wk_ref[...], (((1,), (1,)), ((), ())),
                        preferred_element_type=jnp.float32)
    n = jnp.sqrt(jnp.sum(y * y, axis=-1, keepdims=True))
    o_ref[...] = y / jnp.maximum(n, 1e-12)


_qn_call = pl.pallas_call(
    _qn_body,
    out_shape=jax.ShapeDtypeStruct((_BATCH, _D), jnp.float32),
)


def _proj_body(b_ref, wv_ref, o_ref):
    o_ref[...] = lax.dot_general(b_ref[...], wv_ref[...],
                                 (((1,), (1,)), ((), ())),
                                 preferred_element_type=jnp.float32)


_proj_call = pl.pallas_call(
    _proj_body,
    out_shape=jax.ShapeDtypeStruct((_BATCH, _D), jnp.float32),
)


def kernel(query, keys_buf, values_buf, W_key, W_val, episodic_scale,
           user_ids, memory_count):
    qn = _qn_call(query, W_key)
    uid = user_ids.astype(jnp.int32)
    cnt = memory_count.astype(jnp.int32)
    keysT = jnp.transpose(keys_buf, (1, 2, 0))
    valsT = jnp.transpose(values_buf, (1, 2, 0))
    uk, uv = _extract(keysT, valsT, uid)
    blended = _retrieve(qn.reshape(-1), uk, uv, uid, cnt)
    blended = blended.reshape(_BATCH, _D)
    return _proj_call(blended, W_val * episodic_scale)
